# baseline (speedup 1.0000x reference)
import functools

import jax
import jax.numpy as jnp
from jax import lax
from jax.experimental import pallas as pl
from jax.experimental.pallas import tpu as pltpu
from jax.experimental.pallas import tpu_sc as plsc

BATCH = 4096
EMB = 64
NC = 2
NS = 16
NW = NC * NS
B_PER_W = BATCH // NW
NB = 4
NG = B_PER_W // NB
LANE = 128


def _gather_tab(tabT, idx_v, rows, sems, slabs):
    iota = lax.broadcasted_iota(jnp.int32, (16,), 0)

    def fire(g, bank, sem):
        cols = idx_v[pl.ds(g * NB, 16)]
        for b in range(NB):
            col = cols[b]
            start = pl.multiple_of((col >> 7) << 7, LANE)
            pltpu.async_copy(tabT.at[:, pl.ds(start, LANE)],
                             slabs.at[bank + b], sem)

    def drain_extract(g, bank, sem):
        for _ in range(NB):
            pltpu.make_async_copy(tabT.at[:, pl.ds(0, LANE)], slabs.at[0],
                                  sem).wait()
        cols = idx_v[pl.ds(g * NB, 16)]
        for b in range(NB):
            lane = cols[b] & (LANE - 1)
            sub = jnp.broadcast_to(lane & 15, (16,))
            cstart = (lane >> 4) << 4
            j = g * NB + b
            for q in range(EMB // 16):
                acc = jnp.zeros((16,), jnp.float32)
                for f16 in range(16):
                    f = q * 16 + f16
                    v16 = slabs[bank + b, f, pl.ds(cstart, 16)]
                    val = jnp.take(v16, sub)
                    acc = jnp.where(iota == f16, val, acc)
                rows[pl.ds(j, 1), pl.ds(q * 16, 16)] = acc.reshape(1, 16)

    fire(0, 0, sems[0])
    fire(1, NB, sems[1])
    fire(2, 2 * NB, sems[2])

    def body(h, carry):
        g = 3 * h
        drain_extract(g, 0, sems[0])
        fire(g + 3, 0, sems[0])
        drain_extract(g + 1, NB, sems[1])
        fire(g + 4, NB, sems[1])
        drain_extract(g + 2, 2 * NB, sems[2])
        fire(g + 5, 2 * NB, sems[2])
        return carry

    lax.fori_loop(0, (NG - 5) // 3, body, 0)
    drain_extract(NG - 5, 0, sems[0])
    fire(NG - 2, 0, sems[0])
    drain_extract(NG - 4, NB, sems[1])
    fire(NG - 1, NB, sems[1])
    drain_extract(NG - 3, 2 * NB, sems[2])
    drain_extract(NG - 2, 0, sems[0])
    drain_extract(NG - 1, NB, sems[1])


def _user_body(user_tabT, uidx, uout, uidx_v, urows, slabs,
               sem0, sem1, sem2):
    wid = lax.axis_index("s") * NC + lax.axis_index("c")
    base = wid * B_PER_W
    pltpu.sync_copy(uidx.at[pl.ds(base, B_PER_W)],
                    uidx_v.at[pl.ds(0, B_PER_W)])
    _gather_tab(user_tabT, uidx_v, urows, (sem0, sem1, sem2), slabs)
    pltpu.sync_copy(urows, uout.at[pl.ds(base, B_PER_W)])


def _sc_gather_user(user, user_tableT):
    mesh = plsc.VectorSubcoreMesh(core_axis_name="c", subcore_axis_name="s",
                                  num_cores=NC, num_subcores=NS)
    return pl.kernel(
        _user_body,
        out_type=jax.ShapeDtypeStruct((BATCH, EMB), jnp.float32),
        mesh=mesh,
        scratch_types=[
            pltpu.VMEM((B_PER_W + 16,), jnp.int32),
            pltpu.VMEM((B_PER_W, EMB), jnp.float32),
            pltpu.VMEM((3 * NB, EMB, LANE), jnp.float32),
            pltpu.SemaphoreType.DMA,
            pltpu.SemaphoreType.DMA,
            pltpu.SemaphoreType.DMA,
        ],
        compiler_params=pltpu.CompilerParams(disable_bounds_checks=True),
    )(user_tableT, user)


def _item_body(item_tab, iidx, iout, iidx_v, irows, sem):
    wid = lax.axis_index("s") * NC + lax.axis_index("c")
    base = wid * B_PER_W
    pltpu.sync_copy(iidx.at[pl.ds(base, B_PER_W)],
                    iidx_v.at[pl.ds(0, B_PER_W)])

    def fire(c, idx_v, tab, rows):
        chunk = idx_v[pl.ds(c * 16, 16)]
        for j in range(16):
            row = chunk[j]
            pltpu.async_copy(tab.at[pl.ds(row, 1)],
                             rows.at[pl.ds(c * 16 + j, 1)], sem)

    def body(c, carry):
        fire(c, iidx_v, item_tab, irows)
        return carry

    lax.fori_loop(0, B_PER_W // 16, body, 0, unroll=True)
    pltpu.make_async_copy(item_tab.at[pl.ds(0, B_PER_W)], irows, sem).wait()
    pltpu.sync_copy(irows, iout.at[pl.ds(base, B_PER_W)])


def _sc_gather_item(item, item_table):
    mesh = plsc.VectorSubcoreMesh(core_axis_name="c", subcore_axis_name="s",
                                  num_cores=NC, num_subcores=NS)
    return pl.kernel(
        _item_body,
        out_type=jax.ShapeDtypeStruct((BATCH, EMB), jnp.float32),
        mesh=mesh,
        scratch_types=[
            pltpu.VMEM((B_PER_W + 16,), jnp.int32),
            pltpu.VMEM((B_PER_W, EMB), jnp.float32),
            pltpu.SemaphoreType.DMA,
        ],
        compiler_params=pltpu.CompilerParams(disable_bounds_checks=True),
    )(item_table, item)


def _dense_body(ue_ref, ie_ref, w1u_ref, w1i_ref, b1_ref, w2_ref, b2_ref,
                w3_ref, b3_ref, gmf_w_ref, wf_h_ref, fbias_ref, out_ref):
    ue = ue_ref[...]
    ie = ie_ref[...]
    h = jnp.maximum(
        jnp.dot(ue, w1u_ref[...], preferred_element_type=jnp.float32)
        + jnp.dot(ie, w1i_ref[...], preferred_element_type=jnp.float32)
        + b1_ref[...], 0.0)
    h = jnp.maximum(
        jnp.dot(h, w2_ref[...], preferred_element_type=jnp.float32)
        + b2_ref[...], 0.0)
    h = jnp.maximum(
        jnp.dot(h, w3_ref[...], preferred_element_type=jnp.float32)
        + b3_ref[...], 0.0)
    gmf = lax.dot_general(gmf_w_ref[...], ue * ie, (((1,), (1,)), ((), ())),
                          preferred_element_type=jnp.float32)
    z = (lax.dot_general(wf_h_ref[...], h, (((1,), (1,)), ((), ())),
                         preferred_element_type=jnp.float32)
         + gmf * fbias_ref[0, 0] + fbias_ref[0, 1])
    out_ref[...] = (1.0 / (1.0 + jnp.exp(-z))).reshape(z.shape[1])


_DB = 512


def _tc_dense(ue, ie, w1u, w1i, b1, w2, b2, w3, b3, gmf_w_row, wf_h_row,
              fbias):
    full = lambda shape: pl.BlockSpec(shape, lambda i: (0,) * len(shape))
    return pl.pallas_call(
        _dense_body,
        grid=(BATCH // _DB,),
        in_specs=[
            pl.BlockSpec((_DB, EMB), lambda i: (i, 0)),
            pl.BlockSpec((_DB, EMB), lambda i: (i, 0)),
            full((EMB, 128)), full((EMB, 128)), full((1, 128)),
            full((128, EMB)), full((1, EMB)),
            full((EMB, 32)), full((1, 32)),
            full((1, EMB)), full((1, 32)), full((1, 2)),
        ],
        out_specs=pl.BlockSpec((_DB,), lambda i: (i,)),
        out_shape=jax.ShapeDtypeStruct((BATCH,), jnp.float32),
    )(ue, ie, w1u, w1i, b1, w2, b2, w3, b3, gmf_w_row, wf_h_row, fbias)


def kernel(user, item, user_table, item_table, gmf_W, gmf_b,
           fc1_W, fc1_b, fc2_W, fc2_b, fc3_W, fc3_b, final_W, final_b):
    ue = _sc_gather_user(user.astype(jnp.int32), user_table.T)
    dep = (ue[0, 0] * 0.0).astype(jnp.int32)
    ie = _sc_gather_item(item.astype(jnp.int32) + dep, item_table)
    w1u = fc1_W[:EMB]
    w1i = fc1_W[EMB:]
    b1 = fc1_b.reshape(1, -1)
    b2 = fc2_b.reshape(1, -1)
    b3 = fc3_b.reshape(1, -1)
    gmf_w_row = gmf_W.reshape(1, EMB)
    wf_h_row = final_W[1:, 0].reshape(1, 32)
    w0 = final_W[0, 0]
    fbias = jnp.stack([w0, final_b[0] + gmf_b[0] * w0]).reshape(1, 2)
    return _tc_dense(ue, ie, w1u, w1i, b1, fc2_W, b2, fc3_W, b3,
                     gmf_w_row, wf_h_row, fbias)


# speedup vs baseline: 1.0345x; 1.0345x over previous
import functools

import jax
import jax.numpy as jnp
from jax import lax
from jax.experimental import pallas as pl
from jax.experimental.pallas import tpu as pltpu
from jax.experimental.pallas import tpu_sc as plsc

BATCH = 4096
EMB = 64
NC = 2
NS = 16
NW = NC * NS
B_PER_W = BATCH // NW
NB = 4
NG = B_PER_W // NB
LANE = 128


def _gather_tab(tabT, idx_v, rows, sems, slabs):
    iota = lax.broadcasted_iota(jnp.int32, (16,), 0)

    def fire(g, bank, sem):
        cols = idx_v[pl.ds(g * NB, 16)]
        for b in range(NB):
            col = cols[b]
            start = pl.multiple_of((col >> 7) << 7, LANE)
            pltpu.async_copy(tabT.at[:, pl.ds(start, LANE)],
                             slabs.at[bank + b], sem)

    def drain_extract(g, bank, sem):
        for _ in range(NB):
            pltpu.make_async_copy(tabT.at[:, pl.ds(0, LANE)], slabs.at[0],
                                  sem).wait()
        cols = idx_v[pl.ds(g * NB, 16)]
        for b in range(NB):
            lane = cols[b] & (LANE - 1)
            sub = jnp.broadcast_to(lane & 15, (16,))
            cstart = (lane >> 4) << 4
            j = g * NB + b
            for q in range(EMB // 16):
                acc = jnp.zeros((16,), jnp.float32)
                for f16 in range(16):
                    f = q * 16 + f16
                    v16 = slabs[bank + b, f, pl.ds(cstart, 16)]
                    val = jnp.take(v16, sub)
                    acc = jnp.where(iota == f16, val, acc)
                rows[pl.ds(j, 1), pl.ds(q * 16, 16)] = acc.reshape(1, 16)

    fire(0, 0, sems[0])
    fire(1, NB, sems[1])
    fire(2, 2 * NB, sems[2])

    def body(h, carry):
        g = 3 * h
        drain_extract(g, 0, sems[0])
        fire(g + 3, 0, sems[0])
        drain_extract(g + 1, NB, sems[1])
        fire(g + 4, NB, sems[1])
        drain_extract(g + 2, 2 * NB, sems[2])
        fire(g + 5, 2 * NB, sems[2])
        return carry

    lax.fori_loop(0, (NG - 5) // 3, body, 0)
    drain_extract(NG - 5, 0, sems[0])
    fire(NG - 2, 0, sems[0])
    drain_extract(NG - 4, NB, sems[1])
    fire(NG - 1, NB, sems[1])
    drain_extract(NG - 3, 2 * NB, sems[2])
    drain_extract(NG - 2, 0, sems[0])
    drain_extract(NG - 1, NB, sems[1])


def _user_body(user_tabT, uidx, uout, uidx_v, urows, slabs,
               sem0, sem1, sem2):
    wid = lax.axis_index("s") * NC + lax.axis_index("c")
    base = wid * B_PER_W
    pltpu.sync_copy(uidx.at[pl.ds(base, B_PER_W)],
                    uidx_v.at[pl.ds(0, B_PER_W)])
    _gather_tab(user_tabT, uidx_v, urows, (sem0, sem1, sem2), slabs)
    pltpu.sync_copy(urows, uout.at[pl.ds(base, B_PER_W)])


def _sc_gather_user(user, user_tableT):
    mesh = plsc.VectorSubcoreMesh(core_axis_name="c", subcore_axis_name="s",
                                  num_cores=NC, num_subcores=NS)
    return pl.kernel(
        _user_body,
        out_type=jax.ShapeDtypeStruct((BATCH, EMB), jnp.float32),
        mesh=mesh,
        scratch_types=[
            pltpu.VMEM((B_PER_W + 16,), jnp.int32),
            pltpu.VMEM((B_PER_W, EMB), jnp.float32),
            pltpu.VMEM((3 * NB, EMB, LANE), jnp.float32),
            pltpu.SemaphoreType.DMA,
            pltpu.SemaphoreType.DMA,
            pltpu.SemaphoreType.DMA,
        ],
        compiler_params=pltpu.CompilerParams(disable_bounds_checks=True),
    )(user_tableT, user)


def _item_body(item_tab, iidx, iout, iidx_v, irows, sem):
    wid = lax.axis_index("s") * NC + lax.axis_index("c")
    base = wid * B_PER_W
    pltpu.sync_copy(iidx.at[pl.ds(base, B_PER_W)],
                    iidx_v.at[pl.ds(0, B_PER_W)])

    def fire(c, idx_v, tab, rows):
        chunk = idx_v[pl.ds(c * 16, 16)]
        for j in range(16):
            row = chunk[j]
            pltpu.async_copy(tab.at[pl.ds(row, 1)],
                             rows.at[pl.ds(c * 16 + j, 1)], sem)

    def body(c, carry):
        fire(c, iidx_v, item_tab, irows)
        return carry

    lax.fori_loop(0, B_PER_W // 16, body, 0, unroll=True)
    pltpu.make_async_copy(item_tab.at[pl.ds(0, B_PER_W)], irows, sem).wait()
    pltpu.sync_copy(irows, iout.at[pl.ds(base, B_PER_W)])


def _sc_gather_item(item, item_table):
    mesh = plsc.VectorSubcoreMesh(core_axis_name="c", subcore_axis_name="s",
                                  num_cores=NC, num_subcores=NS)
    return pl.kernel(
        _item_body,
        out_type=jax.ShapeDtypeStruct((BATCH, EMB), jnp.float32),
        mesh=mesh,
        scratch_types=[
            pltpu.VMEM((B_PER_W + 16,), jnp.int32),
            pltpu.VMEM((B_PER_W, EMB), jnp.float32),
            pltpu.SemaphoreType.DMA,
        ],
        compiler_params=pltpu.CompilerParams(disable_bounds_checks=True),
    )(item_table, item)


def _dense_body(ue_ref, ie_ref, w1u_ref, w1i_ref, b1_ref, w2_ref, b2_ref,
                w3_ref, b3_ref, gmf_w_ref, wf_h_ref, fbias_ref, out_ref):
    ue = ue_ref[...]
    ie = ie_ref[...]
    h = jnp.maximum(
        jnp.dot(ue, w1u_ref[...], preferred_element_type=jnp.float32)
        + jnp.dot(ie, w1i_ref[...], preferred_element_type=jnp.float32)
        + b1_ref[...], 0.0)
    h = jnp.maximum(
        jnp.dot(h, w2_ref[...], preferred_element_type=jnp.float32)
        + b2_ref[...], 0.0)
    h = jnp.maximum(
        jnp.dot(h, w3_ref[...], preferred_element_type=jnp.float32)
        + b3_ref[...], 0.0)
    gmf = lax.dot_general(gmf_w_ref[...], ue * ie, (((1,), (1,)), ((), ())),
                          preferred_element_type=jnp.float32)
    z = (lax.dot_general(wf_h_ref[...], h, (((1,), (1,)), ((), ())),
                         preferred_element_type=jnp.float32)
         + gmf * fbias_ref[0, 0] + fbias_ref[0, 1])
    out_ref[...] = (1.0 / (1.0 + jnp.exp(-z))).reshape(z.shape[1])


_DB = 2048


def _tc_dense(ue, ie, w1u, w1i, b1, w2, b2, w3, b3, gmf_w_row, wf_h_row,
              fbias):
    full = lambda shape: pl.BlockSpec(shape, lambda i: (0,) * len(shape))
    return pl.pallas_call(
        _dense_body,
        grid=(BATCH // _DB,),
        in_specs=[
            pl.BlockSpec((_DB, EMB), lambda i: (i, 0)),
            pl.BlockSpec((_DB, EMB), lambda i: (i, 0)),
            full((EMB, 128)), full((EMB, 128)), full((1, 128)),
            full((128, EMB)), full((1, EMB)),
            full((EMB, 32)), full((1, 32)),
            full((1, EMB)), full((1, 32)), full((1, 2)),
        ],
        out_specs=pl.BlockSpec((_DB,), lambda i: (i,)),
        out_shape=jax.ShapeDtypeStruct((BATCH,), jnp.float32),
    )(ue, ie, w1u, w1i, b1, w2, b2, w3, b3, gmf_w_row, wf_h_row, fbias)


def kernel(user, item, user_table, item_table, gmf_W, gmf_b,
           fc1_W, fc1_b, fc2_W, fc2_b, fc3_W, fc3_b, final_W, final_b):
    ue = _sc_gather_user(user.astype(jnp.int32), user_table.T)
    dep = (ue[0, 0] * 0.0).astype(jnp.int32)
    ie = _sc_gather_item(item.astype(jnp.int32) + dep, item_table)
    w1u = fc1_W[:EMB]
    w1i = fc1_W[EMB:]
    b1 = fc1_b.reshape(1, -1)
    b2 = fc2_b.reshape(1, -1)
    b3 = fc3_b.reshape(1, -1)
    gmf_w_row = gmf_W.reshape(1, EMB)
    wf_h_row = final_W[1:, 0].reshape(1, 32)
    w0 = final_W[0, 0]
    fbias = jnp.stack([w0, final_b[0] + gmf_b[0] * w0]).reshape(1, 2)
    return _tc_dense(ue, ie, w1u, w1i, b1, fc2_W, b2, fc3_W, b3,
                     gmf_w_row, wf_h_row, fbias)
